# baseline (speedup 1.0000x reference)
import jax
import jax.numpy as jnp
from jax.experimental import pallas as pl
from jax.experimental.pallas import tpu as pltpu

_N = 4096
_DOUT = 64
_TM = 128
_TMD = 256
_F32 = jnp.float32
_BF16 = jnp.bfloat16


def _bdot(a, b):
    return jnp.dot(a.astype(_BF16), b.astype(_BF16),
                   preferred_element_type=_F32)


def _p_mlp_w1(P, k):
    return P[64 * k:64 * (k + 1), 0:64]


def _p_wdec(P, k, d):
    return P[64 * k:64 * (k + 1), 64:64 + d]


def _p_ced_w1(P, k):
    return P[0:64, 320 + 32 * k:352 + 32 * k]


def _p_ced_w2(P, k):
    return P[32 * k:32 * (k + 1), 416:480]


def _p_mlp_w2(P):
    return P[96:160, 480:544]


def _p_vec(P, r):
    return P[r:r + 1, 544:608]


def _p_ced_b1(P, k):
    return P[9 + k:10 + k, 544:576]


def _p_scal(P, k):
    return P[16:17, 544 + k:545 + k]


def _prep_body(f1, f2, f3, w1, w2, w3,
               mw1, mw2, wd1, wd2, wd3,
               cw1, cw2, cw3, scb,
               cw11, cw12, cw13, cw21, cw22, cw23,
               lng1, lnb1, lng2, lnb2, lng3, lnb3,
               cb11, cb12, cb13, cb21, cb22, cb23,
               mb1, mb2,
               ofemb, obrow, P_o):
    P_o[0:192, 0:64] = mw1[...]
    P_o[0:64, 64:64 + wd1.shape[1]] = wd1[...]
    P_o[64:128, 64:64 + wd2.shape[1]] = wd2[...]
    P_o[128:192, 64:64 + wd3.shape[1]] = wd3[...]
    for k, (a, b) in enumerate(((cw11, cw21), (cw12, cw22), (cw13, cw23))):
        P_o[0:64, 320 + 32 * k:352 + 32 * k] = a[...]
        P_o[32 * k:32 * (k + 1), 416:480] = b[...]
    P_o[96:160, 480:544] = mw2[...]
    for k, (g, b, b2) in enumerate(((lng1, lnb1, cb21), (lng2, lnb2, cb22),
                                    (lng3, lnb3, cb23))):
        P_o[3 * k:3 * k + 1, 544:608] = g[...]
        P_o[3 * k + 1:3 * k + 2, 544:608] = b[...]
        P_o[3 * k + 2:3 * k + 3, 544:608] = b2[...]
    P_o[9:10, 544:576] = cb11[...]
    P_o[10:11, 544:576] = cb12[...]
    P_o[11:12, 544:576] = cb13[...]
    P_o[12:13, 544:608] = mb1[...]
    P_o[13:14, 544:608] = mb2[...]
    s = scb[...]
    scal_row = jnp.concatenate(
        [cw1[...], s[:, 0:1], cw2[...], s[:, 1:2], cw3[...], s[:, 2:3],
         s[:, 3:6], jnp.zeros((1, 52), _F32)], axis=1)
    P_o[16:17, 544:608] = scal_row

    conv = (cw1, cw2, cw3)
    rows = []
    outs = []
    for k, (f, w) in enumerate(((f1, w1), (f2, w2), (f3, w3))):
        femb = _bdot(f[...], w[...])
        outs.append((femb * conv[k][0:1, 0:1]).astype(_BF16))
        outs.append((femb * conv[k][0:1, 1:2]).astype(_BF16))
        rows.append(jnp.sum(femb, axis=0, keepdims=True) * s[0:1, k:k + 1])
    ofemb[...] = jnp.concatenate(outs, axis=1)
    obrow[...] = jnp.concatenate(rows + [jnp.zeros((5, _DOUT), _F32)], axis=0)


def _enc_body(asp1, aft1, asp2, aft2, asp3, aft3, fembp, brow,
              g1_o, g2_o, g3_o):
    fe = fembp[...]
    br = brow[...]

    def one(k, asp, aft):
        return (jnp.dot(asp[...].astype(_BF16), fe[:, 128 * k:128 * k + 64],
                        preferred_element_type=_F32)
                + jnp.dot(aft[...].astype(_BF16),
                          fe[:, 128 * k + 64:128 * k + 128],
                          preferred_element_type=_F32)
                + br[k:k + 1, :])

    g1_o[...] = one(0, asp1, aft1)
    g2_o[...] = one(1, asp2, aft2)
    g3_o[...] = one(2, asp3, aft3)


def _mid_body(g1, g2, g3, P_ref, lat1_o, lat2_o, lat3_o, comb_o):
    P = P_ref[...]

    def ced(k, gco):
        mu = jnp.mean(gco, axis=-1, keepdims=True)
        var = jnp.mean((gco - mu) ** 2, axis=-1, keepdims=True)
        nx = ((gco - mu) / jnp.sqrt(var + 1e-5) * _p_vec(P, 3 * k)
              + _p_vec(P, 3 * k + 1))
        h = jnp.maximum(_bdot(nx, _p_ced_w1(P, k)) + _p_ced_b1(P, k), 0.0)
        enh = _bdot(h, _p_ced_w2(P, k)) + _p_vec(P, 3 * k + 2)
        return gco + _p_scal(P, 9 + k) * enh

    l1 = ced(0, g1[...])
    l2 = ced(1, g2[...])
    l3 = ced(2, g3[...])
    lat1_o[...] = l1
    lat2_o[...] = l2
    lat3_o[...] = l3
    t = (_bdot(l1, _p_mlp_w1(P, 0)) + _bdot(l2, _p_mlp_w1(P, 1))
         + _bdot(l3, _p_mlp_w1(P, 2)) + _p_vec(P, 12))
    comb_o[...] = _bdot(t, _p_mlp_w2(P)) + _p_vec(P, 13)


def _dec_body(asp1, asp2, asp3, comb, P_ref, r1, r2, r3):
    P = P_ref[...]
    cb = comb[...].astype(_BF16)
    d1 = r1.shape[1]
    d2 = r2.shape[1]
    d3 = r3.shape[1]
    t1 = jnp.dot(asp1[...].astype(_BF16), cb, preferred_element_type=_F32)
    t2 = jnp.dot(asp2[...].astype(_BF16), cb, preferred_element_type=_F32)
    t3 = jnp.dot(asp3[...].astype(_BF16), cb, preferred_element_type=_F32)
    r1[...] = _bdot(t1, _p_wdec(P, 0, d1))
    r2[...] = _bdot(t2, _p_wdec(P, 1, d2))
    r3[...] = _bdot(t3, _p_wdec(P, 2, d3))


def _full(shape):
    return pl.BlockSpec(shape, lambda i: (0, 0))


def _rows(tm, cols):
    return pl.BlockSpec((tm, cols), lambda i: (i, 0))


def kernel(features_omics1, features_omics2, features_omics3,
           adj_spatial_omics1, adj_feature_omics1,
           adj_spatial_omics2, adj_feature_omics2,
           adj_spatial_omics3, adj_feature_omics3,
           conv1_w, conv1_b, conv2_w, conv2_b, conv3_w, conv3_b,
           W_enc1, W_enc2, W_enc3,
           ced1_ln_g, ced1_ln_b, ced1_w1, ced1_b1, ced1_w2, ced1_b2,
           ced1_alpha,
           ced2_ln_g, ced2_ln_b, ced2_w1, ced2_b1, ced2_w2, ced2_b2,
           ced2_alpha,
           ced3_ln_g, ced3_ln_b, ced3_w1, ced3_b1, ced3_w2, ced3_b2,
           ced3_alpha,
           mlp_w1, mlp_b1, mlp_w2, mlp_b2,
           W_dec1, W_dec2, W_dec3):
    f32 = _F32
    d1 = features_omics1.shape[1]
    d2 = features_omics2.shape[1]
    d3 = features_omics3.shape[1]

    r2d = lambda a, shp: jnp.reshape(a, shp)
    scb = r2d(jnp.stack([conv1_b, conv2_b, conv3_b,
                         ced1_alpha, ced2_alpha, ced3_alpha,
                         jnp.zeros((), f32), jnp.zeros((), f32)]), (1, 8))

    fembp, brow, P = pl.pallas_call(
        _prep_body,
        out_shape=[jax.ShapeDtypeStruct((_N, 6 * _DOUT), _BF16),
                   jax.ShapeDtypeStruct((8, _DOUT), f32),
                   jax.ShapeDtypeStruct((192, 1024), f32)],
    )(features_omics1, features_omics2, features_omics3,
      W_enc1, W_enc2, W_enc3,
      mlp_w1, mlp_w2, W_dec1, W_dec2, W_dec3,
      r2d(conv1_w, (1, 2)), r2d(conv2_w, (1, 2)), r2d(conv3_w, (1, 2)), scb,
      ced1_w1, ced2_w1, ced3_w1, ced1_w2, ced2_w2, ced3_w2,
      r2d(ced1_ln_g, (1, 64)), r2d(ced1_ln_b, (1, 64)),
      r2d(ced2_ln_g, (1, 64)), r2d(ced2_ln_b, (1, 64)),
      r2d(ced3_ln_g, (1, 64)), r2d(ced3_ln_b, (1, 64)),
      r2d(ced1_b1, (1, 32)), r2d(ced2_b1, (1, 32)), r2d(ced3_b1, (1, 32)),
      r2d(ced1_b2, (1, 64)), r2d(ced2_b2, (1, 64)), r2d(ced3_b2, (1, 64)),
      r2d(mlp_b1, (1, 64)), r2d(mlp_b2, (1, 64)))

    nb = _N // _TM
    gshape = jax.ShapeDtypeStruct((_N, _DOUT), f32)
    g1, g2, g3 = pl.pallas_call(
        _enc_body,
        grid=(nb,),
        in_specs=[_rows(_TM, _N)] * 6
        + [_full((_N, 6 * _DOUT)), _full((8, _DOUT))],
        out_specs=[_rows(_TM, _DOUT)] * 3,
        out_shape=[gshape] * 3,
        compiler_params=pltpu.CompilerParams(
            dimension_semantics=("arbitrary",)),
    )(adj_spatial_omics1, adj_feature_omics1,
      adj_spatial_omics2, adj_feature_omics2,
      adj_spatial_omics3, adj_feature_omics3,
      fembp, brow)

    lat1, lat2, lat3, comb = pl.pallas_call(
        _mid_body,
        out_shape=[gshape] * 4,
    )(g1, g2, g3, P)

    nbd = _N // _TMD
    rec1, rec2, rec3 = pl.pallas_call(
        _dec_body,
        grid=(nbd,),
        in_specs=[_rows(_TMD, _N)] * 3
        + [_full((_N, _DOUT)), _full((192, 1024))],
        out_specs=[_rows(_TMD, d1), _rows(_TMD, d2), _rows(_TMD, d3)],
        out_shape=[jax.ShapeDtypeStruct((_N, d1), f32),
                   jax.ShapeDtypeStruct((_N, d2), f32),
                   jax.ShapeDtypeStruct((_N, d3), f32)],
        compiler_params=pltpu.CompilerParams(
            dimension_semantics=("arbitrary",)),
    )(adj_spatial_omics1, adj_spatial_omics2, adj_spatial_omics3,
      comb, P)

    return (lat1, lat2, lat3, comb, rec1, rec2, rec3)


# speedup vs baseline: 1.0341x; 1.0341x over previous
import jax
import jax.numpy as jnp
from jax.experimental import pallas as pl
from jax.experimental.pallas import tpu as pltpu

_N = 4096
_H = _N // 2
_DOUT = 64
_TM = 128
_TMD = 256
_F32 = jnp.float32
_BF16 = jnp.bfloat16


def _bdot(a, b):
    return jnp.dot(a.astype(_BF16), b.astype(_BF16),
                   preferred_element_type=_F32)


def _p_mlp_w1(P, k):
    return P[64 * k:64 * (k + 1), 0:64]


def _p_wdec(P, k, d):
    return P[64 * k:64 * (k + 1), 64:64 + d]


def _p_ced_w1(P, k):
    return P[0:64, 320 + 32 * k:352 + 32 * k]


def _p_ced_w2(P, k):
    return P[32 * k:32 * (k + 1), 416:480]


def _p_mlp_w2(P):
    return P[96:160, 480:544]


def _p_vec(P, r):
    return P[r:r + 1, 544:608]


def _p_ced_b1(P, k):
    return P[9 + k:10 + k, 544:576]


def _p_scal(P, k):
    return P[16:17, 544 + k:545 + k]


def _enc_body(f1, f2, f3, w1, w2, w3,
              mw1, mw2, wd1, wd2, wd3,
              cw1, cw2, cw3, scb,
              cw11, cw12, cw13, cw21, cw22, cw23,
              lng1, lnb1, lng2, lnb2, lng3, lnb3,
              cb11, cb12, cb13, cb21, cb22, cb23,
              mb1, mb2,
              asp1l, asp1r, aft1l, aft1r,
              asp2l, asp2r, aft2l, aft2r,
              asp3l, asp3r, aft3l, aft3r,
              g1_o, g2_o, g3_o, P_o,
              fe_s, br_s):
    i = pl.program_id(0)

    @pl.when(i == 0)
    def _prep():
        P_o[0:192, 0:64] = mw1[...]
        P_o[0:64, 64:64 + wd1.shape[1]] = wd1[...]
        P_o[64:128, 64:64 + wd2.shape[1]] = wd2[...]
        P_o[128:192, 64:64 + wd3.shape[1]] = wd3[...]
        for k, (a, b) in enumerate(((cw11, cw21), (cw12, cw22),
                                    (cw13, cw23))):
            P_o[0:64, 320 + 32 * k:352 + 32 * k] = a[...]
            P_o[32 * k:32 * (k + 1), 416:480] = b[...]
        P_o[96:160, 480:544] = mw2[...]
        for k, (g, b, b2) in enumerate(((lng1, lnb1, cb21),
                                        (lng2, lnb2, cb22),
                                        (lng3, lnb3, cb23))):
            P_o[3 * k:3 * k + 1, 544:608] = g[...]
            P_o[3 * k + 1:3 * k + 2, 544:608] = b[...]
            P_o[3 * k + 2:3 * k + 3, 544:608] = b2[...]
        P_o[9:10, 544:576] = cb11[...]
        P_o[10:11, 544:576] = cb12[...]
        P_o[11:12, 544:576] = cb13[...]
        P_o[12:13, 544:608] = mb1[...]
        P_o[13:14, 544:608] = mb2[...]
        s = scb[...]
        P_o[16:17, 544:608] = jnp.concatenate(
            [cw1[...], s[:, 0:1], cw2[...], s[:, 1:2], cw3[...], s[:, 2:3],
             s[:, 3:6], jnp.zeros((1, 52), _F32)], axis=1)
        conv = (cw1, cw2, cw3)
        rows = []
        outs = []
        for k, (f, w) in enumerate(((f1, w1), (f2, w2), (f3, w3))):
            femb = _bdot(f[...], w[...])
            outs.append((femb * conv[k][0:1, 0:1]).astype(_BF16))
            outs.append((femb * conv[k][0:1, 1:2]).astype(_BF16))
            rows.append(jnp.sum(femb, axis=0, keepdims=True)
                        * s[0:1, k:k + 1])
        fe_s[...] = jnp.concatenate(outs, axis=1)
        br_s[...] = jnp.concatenate(rows + [jnp.zeros((5, _DOUT), _F32)],
                                    axis=0)

    fe = fe_s[...]
    br = br_s[...]

    def one(k, aspl, aspr, aftl, aftr):
        c0 = 128 * k
        return (jnp.dot(aspl[...].astype(_BF16), fe[0:_H, c0:c0 + 64],
                        preferred_element_type=_F32)
                + jnp.dot(aspr[...].astype(_BF16), fe[_H:_N, c0:c0 + 64],
                          preferred_element_type=_F32)
                + jnp.dot(aftl[...].astype(_BF16), fe[0:_H, c0 + 64:c0 + 128],
                          preferred_element_type=_F32)
                + jnp.dot(aftr[...].astype(_BF16), fe[_H:_N, c0 + 64:c0 + 128],
                          preferred_element_type=_F32)
                + br[k:k + 1, :])

    g1_o[...] = one(0, asp1l, asp1r, aft1l, aft1r)
    g2_o[...] = one(1, asp2l, asp2r, aft2l, aft2r)
    g3_o[...] = one(2, asp3l, asp3r, aft3l, aft3r)


def _dec_body(g1, g2, g3, P_ref, asp1, asp2, asp3,
              lat1_o, lat2_o, lat3_o, comb_o, r1, r2, r3,
              cb_s):
    j = pl.program_id(0)
    P = P_ref[...]

    @pl.when(j == 0)
    def _mid():
        def ced(k, gco):
            mu = jnp.mean(gco, axis=-1, keepdims=True)
            var = jnp.mean((gco - mu) ** 2, axis=-1, keepdims=True)
            nx = ((gco - mu) / jnp.sqrt(var + 1e-5) * _p_vec(P, 3 * k)
                  + _p_vec(P, 3 * k + 1))
            h = jnp.maximum(_bdot(nx, _p_ced_w1(P, k)) + _p_ced_b1(P, k),
                            0.0)
            enh = _bdot(h, _p_ced_w2(P, k)) + _p_vec(P, 3 * k + 2)
            return gco + _p_scal(P, 9 + k) * enh

        l1 = ced(0, g1[...])
        l2 = ced(1, g2[...])
        l3 = ced(2, g3[...])
        lat1_o[...] = l1
        lat2_o[...] = l2
        lat3_o[...] = l3
        t = (_bdot(l1, _p_mlp_w1(P, 0)) + _bdot(l2, _p_mlp_w1(P, 1))
             + _bdot(l3, _p_mlp_w1(P, 2)) + _p_vec(P, 12))
        comb = _bdot(t, _p_mlp_w2(P)) + _p_vec(P, 13)
        comb_o[...] = comb
        cb_s[...] = comb.astype(_BF16)

    cb = cb_s[...]
    d1 = r1.shape[1]
    d2 = r2.shape[1]
    d3 = r3.shape[1]
    t1 = jnp.dot(asp1[...].astype(_BF16), cb, preferred_element_type=_F32)
    t2 = jnp.dot(asp2[...].astype(_BF16), cb, preferred_element_type=_F32)
    t3 = jnp.dot(asp3[...].astype(_BF16), cb, preferred_element_type=_F32)
    r1[...] = _bdot(t1, _p_wdec(P, 0, d1))
    r2[...] = _bdot(t2, _p_wdec(P, 1, d2))
    r3[...] = _bdot(t3, _p_wdec(P, 2, d3))


def _full(shape):
    return pl.BlockSpec(shape, lambda i: tuple(0 for _ in shape))


def _rows(tm, cols):
    return pl.BlockSpec((tm, cols), lambda i: (i, 0))


def _rowhalf(tm, h):
    return [pl.BlockSpec((tm, _H), lambda i: (i, 0)),
            pl.BlockSpec((tm, _H), lambda i: (i, 1))]


def kernel(features_omics1, features_omics2, features_omics3,
           adj_spatial_omics1, adj_feature_omics1,
           adj_spatial_omics2, adj_feature_omics2,
           adj_spatial_omics3, adj_feature_omics3,
           conv1_w, conv1_b, conv2_w, conv2_b, conv3_w, conv3_b,
           W_enc1, W_enc2, W_enc3,
           ced1_ln_g, ced1_ln_b, ced1_w1, ced1_b1, ced1_w2, ced1_b2,
           ced1_alpha,
           ced2_ln_g, ced2_ln_b, ced2_w1, ced2_b1, ced2_w2, ced2_b2,
           ced2_alpha,
           ced3_ln_g, ced3_ln_b, ced3_w1, ced3_b1, ced3_w2, ced3_b2,
           ced3_alpha,
           mlp_w1, mlp_b1, mlp_w2, mlp_b2,
           W_dec1, W_dec2, W_dec3):
    f32 = _F32
    d1 = features_omics1.shape[1]
    d2 = features_omics2.shape[1]
    d3 = features_omics3.shape[1]

    r2d = lambda a, shp: jnp.reshape(a, shp)
    scb = r2d(jnp.stack([conv1_b, conv2_b, conv3_b,
                         ced1_alpha, ced2_alpha, ced3_alpha,
                         jnp.zeros((), f32), jnp.zeros((), f32)]), (1, 8))

    nb = _N // _TM
    gshape = jax.ShapeDtypeStruct((_N, _DOUT), f32)
    small_specs = [
        _full((_N, d1)), _full((_N, d2)), _full((_N, d3)),
        _full((d1, _DOUT)), _full((d2, _DOUT)), _full((d3, _DOUT)),
        _full((3 * _DOUT, _DOUT)), _full((_DOUT, _DOUT)),
        _full((_DOUT, d1)), _full((_DOUT, d2)), _full((_DOUT, d3)),
        _full((1, 2)), _full((1, 2)), _full((1, 2)), _full((1, 8)),
        _full((_DOUT, 32)), _full((_DOUT, 32)), _full((_DOUT, 32)),
        _full((32, _DOUT)), _full((32, _DOUT)), _full((32, _DOUT)),
    ] + [_full((1, 64))] * 6 + [_full((1, 32))] * 3 + [_full((1, 64))] * 5
    adj_specs = (_rowhalf(_TM, _H) * 6)

    g1, g2, g3, P = pl.pallas_call(
        _enc_body,
        grid=(nb,),
        in_specs=small_specs + adj_specs,
        out_specs=[_rows(_TM, _DOUT)] * 3 + [_full((192, 1024))],
        out_shape=[gshape] * 3 + [jax.ShapeDtypeStruct((192, 1024), f32)],
        scratch_shapes=[pltpu.VMEM((_N, 6 * _DOUT), _BF16),
                        pltpu.VMEM((8, _DOUT), _F32)],
        compiler_params=pltpu.CompilerParams(
            dimension_semantics=("arbitrary",)),
    )(features_omics1, features_omics2, features_omics3,
      W_enc1, W_enc2, W_enc3,
      mlp_w1, mlp_w2, W_dec1, W_dec2, W_dec3,
      r2d(conv1_w, (1, 2)), r2d(conv2_w, (1, 2)), r2d(conv3_w, (1, 2)), scb,
      ced1_w1, ced2_w1, ced3_w1, ced1_w2, ced2_w2, ced3_w2,
      r2d(ced1_ln_g, (1, 64)), r2d(ced1_ln_b, (1, 64)),
      r2d(ced2_ln_g, (1, 64)), r2d(ced2_ln_b, (1, 64)),
      r2d(ced3_ln_g, (1, 64)), r2d(ced3_ln_b, (1, 64)),
      r2d(ced1_b1, (1, 32)), r2d(ced2_b1, (1, 32)), r2d(ced3_b1, (1, 32)),
      r2d(ced1_b2, (1, 64)), r2d(ced2_b2, (1, 64)), r2d(ced3_b2, (1, 64)),
      r2d(mlp_b1, (1, 64)), r2d(mlp_b2, (1, 64)),
      adj_spatial_omics1, adj_spatial_omics1,
      adj_feature_omics1, adj_feature_omics1,
      adj_spatial_omics2, adj_spatial_omics2,
      adj_feature_omics2, adj_feature_omics2,
      adj_spatial_omics3, adj_spatial_omics3,
      adj_feature_omics3, adj_feature_omics3)

    nbd = _N // _TMD
    lat1, lat2, lat3, comb, rec1, rec2, rec3 = pl.pallas_call(
        _dec_body,
        grid=(nbd,),
        in_specs=[_full((_N, _DOUT))] * 3 + [_full((192, 1024))]
        + [_rows(_TMD, _N)] * 3,
        out_specs=[_full((_N, _DOUT))] * 4
        + [_rows(_TMD, d1), _rows(_TMD, d2), _rows(_TMD, d3)],
        out_shape=[gshape] * 4
        + [jax.ShapeDtypeStruct((_N, d1), f32),
           jax.ShapeDtypeStruct((_N, d2), f32),
           jax.ShapeDtypeStruct((_N, d3), f32)],
        scratch_shapes=[pltpu.VMEM((_N, _DOUT), _BF16)],
        compiler_params=pltpu.CompilerParams(
            dimension_semantics=("arbitrary",)),
    )(g1, g2, g3, P,
      adj_spatial_omics1, adj_spatial_omics2, adj_spatial_omics3)

    return (lat1, lat2, lat3, comb, rec1, rec2, rec3)
